# Initial kernel scaffold; baseline (speedup 1.0000x reference)
import jax
import jax.numpy as jnp
from jax.experimental import pallas as pl


def kernel(tensor, table_fix, table_v):
    raise NotImplementedError("write your pallas kernel here")


# speedup vs baseline: 8.1567x; 8.1567x over previous
import jax
import jax.numpy as jnp
from jax import lax
from jax.experimental import pallas as pl
from jax.experimental.pallas import tpu as pltpu
from jax.experimental.pallas import tpu_sc as plsc

_VOCAB = 100000
_DIM = 64
_NC = 2
_NS = 16
_NW = _NC * _NS

_ROWS = 4096 * 200
_RPW = _ROWS // _NW
_CHUNK = 800
_NCHUNK = _RPW // _CHUNK


def _gather_body(idx_hbm, tab_hbm, out_hbm, idx_v, rows_v, sem):
    wid = lax.axis_index("s") * _NC + lax.axis_index("c")
    base = wid * _RPW

    def step(k, carry):
        off = base + k * _CHUNK
        pltpu.sync_copy(idx_hbm.at[pl.ds(off, _CHUNK)], idx_v)
        pltpu.async_copy(tab_hbm.at[idx_v], rows_v, sem).wait()
        pltpu.sync_copy(rows_v, out_hbm.at[pl.ds(off, _CHUNK)])
        return carry

    lax.fori_loop(0, _NCHUNK, step, 0)


_mesh = plsc.VectorSubcoreMesh(core_axis_name="c", subcore_axis_name="s")


def kernel(tensor, table_fix, table_v):
    idx = jnp.swapaxes(tensor, 0, 1).astype(jnp.int32).reshape(_ROWS)
    tab = table_fix.at[_VOCAB - 3:].add(table_v)
    call = pl.kernel(
        _gather_body,
        out_type=jax.ShapeDtypeStruct((_ROWS, _DIM), jnp.float32),
        mesh=_mesh,
        scratch_types=[
            pltpu.VMEM((_CHUNK,), jnp.int32),
            pltpu.VMEM((_CHUNK, _DIM), jnp.float32),
            pltpu.SemaphoreType.DMA,
        ],
        compiler_params=pltpu.CompilerParams(use_tc_tiling_on_sc=False),
    )
    out = call(idx, tab)
    return out.reshape(200, 4096, _DIM)


# speedup vs baseline: 8.4958x; 1.0416x over previous
import jax
import jax.numpy as jnp
from jax import lax
from jax.experimental import pallas as pl
from jax.experimental.pallas import tpu as pltpu
from jax.experimental.pallas import tpu_sc as plsc

_VOCAB = 100000
_DIM = 64
_NC = 2
_NS = 16
_NW = _NC * _NS

_ROWS = 4096 * 200
_RPW = _ROWS // _NW
_CHUNK = 800
_NCHUNK = _RPW // _CHUNK


def _gather_body(idx_hbm, tab_hbm, out_hbm,
                 idx0, idx1, rows0, rows1, si0, si1, sg0, sg1, sw0, sw1):
    wid = lax.axis_index("s") * _NC + lax.axis_index("c")
    base = wid * _RPW
    idxs, rows = (idx0, idx1), (rows0, rows1)
    si, sg, sw = (si0, si1), (sg0, sg1), (sw0, sw1)

    for b in range(2):
        pltpu.async_copy(idx_hbm.at[pl.ds(base + b * _CHUNK, _CHUNK)],
                         idxs[b], si[b])

    def group(g, carry):
        for b in range(2):
            k = 2 * g + b
            off = base + k * _CHUNK
            pltpu.make_async_copy(idx_hbm.at[pl.ds(base, _CHUNK)],
                                  idxs[b], si[b]).wait()

            @pl.when(g > 0)
            def _():
                pltpu.make_async_copy(rows[b],
                                      out_hbm.at[pl.ds(base, _CHUNK)],
                                      sw[b]).wait()

            pltpu.async_copy(tab_hbm.at[idxs[b]], rows[b], sg[b])
            pltpu.make_async_copy(tab_hbm.at[idxs[b]], rows[b], sg[b]).wait()

            pltpu.async_copy(rows[b], out_hbm.at[pl.ds(off, _CHUNK)], sw[b])

            @pl.when(k + 2 < _NCHUNK)
            def _():
                pltpu.async_copy(
                    idx_hbm.at[pl.ds(off + 2 * _CHUNK, _CHUNK)],
                    idxs[b], si[b])
        return carry

    lax.fori_loop(0, _NCHUNK // 2, group, 0)

    for b in range(2):
        pltpu.make_async_copy(rows[b], out_hbm.at[pl.ds(base, _CHUNK)],
                              sw[b]).wait()


_mesh = plsc.VectorSubcoreMesh(core_axis_name="c", subcore_axis_name="s")


def kernel(tensor, table_fix, table_v):
    idx = jnp.swapaxes(tensor, 0, 1).astype(jnp.int32).reshape(_ROWS)
    tab = table_fix.at[_VOCAB - 3:].add(table_v)
    call = pl.kernel(
        _gather_body,
        out_type=jax.ShapeDtypeStruct((_ROWS, _DIM), jnp.float32),
        mesh=_mesh,
        scratch_types=[
            pltpu.VMEM((_CHUNK,), jnp.int32),
            pltpu.VMEM((_CHUNK,), jnp.int32),
            pltpu.VMEM((_CHUNK, _DIM), jnp.float32),
            pltpu.VMEM((_CHUNK, _DIM), jnp.float32),
            pltpu.SemaphoreType.DMA,
            pltpu.SemaphoreType.DMA,
            pltpu.SemaphoreType.DMA,
            pltpu.SemaphoreType.DMA,
            pltpu.SemaphoreType.DMA,
            pltpu.SemaphoreType.DMA,
        ],
        compiler_params=pltpu.CompilerParams(use_tc_tiling_on_sc=False),
    )
    out = call(idx, tab)
    return out.reshape(200, 4096, _DIM)


# speedup vs baseline: 8.5071x; 1.0013x over previous
import jax
import jax.numpy as jnp
from jax import lax
from jax.experimental import pallas as pl
from jax.experimental.pallas import tpu as pltpu
from jax.experimental.pallas import tpu_sc as plsc

_VOCAB = 100000
_DIM = 64
_NC = 2
_NS = 16
_NW = _NC * _NS

_ROWS = 4096 * 200
_RPW = _ROWS // _NW
_CHUNK = 400
_NCHUNK = _RPW // _CHUNK
_NBUF = 4
_A = 2


def _gather_body(idx_hbm, tab_hbm, out_hbm, *scratch):
    idxs = scratch[0:_NBUF]
    rows = scratch[_NBUF:2 * _NBUF]
    si = scratch[2 * _NBUF:3 * _NBUF]
    sg = scratch[3 * _NBUF:4 * _NBUF]
    sw = scratch[4 * _NBUF:5 * _NBUF]

    wid = lax.axis_index("s") * _NC + lax.axis_index("c")
    base = wid * _RPW

    def fire_idx(c, b):
        pltpu.async_copy(idx_hbm.at[pl.ds(base + c * _CHUNK, _CHUNK)],
                         idxs[b], si[b])

    def wait_idx(b):
        pltpu.make_async_copy(idx_hbm.at[pl.ds(base, _CHUNK)],
                              idxs[b], si[b]).wait()

    def fire_gather(b):
        pltpu.async_copy(tab_hbm.at[idxs[b]], rows[b], sg[b])

    def wait_gather(b):
        pltpu.make_async_copy(tab_hbm.at[idxs[b]], rows[b], sg[b]).wait()

    def fire_wb(c, b):
        pltpu.async_copy(rows[b], out_hbm.at[pl.ds(base + c * _CHUNK,
                                                   _CHUNK)], sw[b])

    def wait_wb(b):
        pltpu.make_async_copy(rows[b], out_hbm.at[pl.ds(base, _CHUNK)],
                              sw[b]).wait()

    for c in range(_A + 1):
        fire_idx(c, c % _NBUF)
    for c in range(_A):
        wait_idx(c % _NBUF)
        fire_gather(c % _NBUF)

    def group(g, carry):
        for b in range(_NBUF):
            k = g * _NBUF + b
            ba = (b + _A) % _NBUF
            bn = (b + _A + 1) % _NBUF

            @pl.when(k + _A < _NCHUNK)
            def _():
                wait_idx(ba)

                @pl.when(k + _A >= _NBUF)
                def _():
                    wait_wb(ba)

                fire_gather(ba)

            wait_gather(b)
            fire_wb(k, b)

            @pl.when(k + _A + 1 < _NCHUNK)
            def _():
                fire_idx(k + _A + 1, bn)
        return carry

    lax.fori_loop(0, _NCHUNK // _NBUF, group, 0)

    for b in range(_NBUF):
        wait_wb(b)


_mesh = plsc.VectorSubcoreMesh(core_axis_name="c", subcore_axis_name="s")


def kernel(tensor, table_fix, table_v):
    idx = jnp.swapaxes(tensor, 0, 1).astype(jnp.int32).reshape(_ROWS)
    tab = table_fix.at[_VOCAB - 3:].add(table_v)
    call = pl.kernel(
        _gather_body,
        out_type=jax.ShapeDtypeStruct((_ROWS, _DIM), jnp.float32),
        mesh=_mesh,
        scratch_types=(
            [pltpu.VMEM((_CHUNK,), jnp.int32) for _ in range(_NBUF)]
            + [pltpu.VMEM((_CHUNK, _DIM), jnp.float32) for _ in range(_NBUF)]
            + [pltpu.SemaphoreType.DMA for _ in range(3 * _NBUF)]
        ),
        compiler_params=pltpu.CompilerParams(use_tc_tiling_on_sc=False),
    )
    out = call(idx, tab)
    return out.reshape(200, 4096, _DIM)
